# baseline (speedup 1.0000x reference)
import functools

import jax
import jax.numpy as jnp
from jax import lax
from jax.experimental import pallas as pl
from jax.experimental.pallas import tpu as pltpu
from jax.experimental.pallas import tpu_sc as plsc

N = 10000
E = 160000
D = 256
EB = 16
HD = 512

NP = 10240
NC = 2
NS = 16
NW = NC * NS
EPW = E // NW
EPT = E // NS

GC = 32
GT = 8
CC = 32
SC_CHUNK = 80
FB = 128
RPT = NP // NS

_mesh = plsc.VectorSubcoreMesh(core_axis_name="c", subcore_axis_name="s")


def _wid():
    return lax.axis_index("s") * NC + lax.axis_index("c")


def _gather_body(table_hbm, idx_hbm, out_hbm, idx_all, rv0, rv1, sg0, sg1,
                 so0, so1):
    base = _wid() * EPW
    pltpu.sync_copy(idx_hbm.at[pl.ds(base, EPW)], idx_all)
    nch = EPW // GC

    def g_issue(j, rv, sg, n=GC):
        pltpu.make_async_copy(
            table_hbm.at[idx_all.at[pl.ds(j * GC, n)]], rv.at[pl.ds(0, n)],
            sg).start()

    def g_wait(j, rv, sg, n=GC):
        pltpu.make_async_copy(
            table_hbm.at[idx_all.at[pl.ds(j * GC, n)]], rv.at[pl.ds(0, n)],
            sg).wait()

    def s_issue(j, rv, so, n=GC):
        pltpu.make_async_copy(
            rv.at[pl.ds(0, n)], out_hbm.at[pl.ds(base + j * GC, n)],
            so).start()

    def s_wait(j, rv, so, n=GC):
        pltpu.make_async_copy(
            rv.at[pl.ds(0, n)], out_hbm.at[pl.ds(base + j * GC, n)],
            so).wait()

    g_issue(0, rv0, sg0)
    g_issue(1, rv1, sg1)

    def pair(k, carry):
        j0 = 2 * k
        g_wait(j0, rv0, sg0)
        s_issue(j0, rv0, so0)
        g_wait(j0 + 1, rv1, sg1)
        s_issue(j0 + 1, rv1, so1)
        s_wait(j0, rv0, so0)

        @pl.when(k < nch // 2 - 1)
        def _():
            g_issue(j0 + 2, rv0, sg0)

        s_wait(j0 + 1, rv1, so1)

        @pl.when(k < nch // 2 - 1)
        def _():
            g_issue(j0 + 3, rv1, sg1)

        return carry

    lax.fori_loop(0, nch // 2, pair, 0)
    g_issue(nch, rv0, sg0, n=GT)
    g_wait(nch, rv0, sg0, n=GT)
    s_issue(nch, rv0, so0, n=GT)
    s_wait(nch, rv0, so0, n=GT)


def _sc_gather(table, idx):
    return pl.kernel(
        _gather_body,
        out_type=jax.ShapeDtypeStruct((E, HD), jnp.float32),
        mesh=_mesh,
        scratch_types=[
            pltpu.VMEM((EPW,), jnp.int32),
            pltpu.VMEM((GC, HD), jnp.float32),
            pltpu.VMEM((GC, HD), jnp.float32),
            pltpu.SemaphoreType.DMA,
            pltpu.SemaphoreType.DMA,
            pltpu.SemaphoreType.DMA,
            pltpu.SemaphoreType.DMA,
        ],
    )(table, idx)


def _scatter_body(p_hbm, dst_hbm, z_hbm, q_hbm, q_sh,
                  ix0, dv0, ix1, dv1, ix2, dv2, ix3, dv3,
                  si0, sd0, si1, sd1, si2, sd2, si3, sd3, sa):
    cid = lax.axis_index("c")
    sid = lax.axis_index("s")
    ebase = sid * EPT
    rbase = sid * RPT
    nch = EPT // SC_CHUNK
    banks = ((ix0, dv0, si0, sd0), (ix1, dv1, si1, sd1),
             (ix2, dv2, si2, sd2), (ix3, dv3, si3, sd3))
    ngrp = nch // 4

    for p in range(2):
        f0 = (cid * 2 + p) * FB

        def l_issue(j, bank):
            ix, dv, si, sd = bank
            b = ebase + j * SC_CHUNK
            pltpu.make_async_copy(dst_hbm.at[pl.ds(b, SC_CHUNK)], ix,
                                  si).start()
            pltpu.make_async_copy(
                p_hbm.at[pl.ds(b, SC_CHUNK), pl.ds(f0, FB)], dv, sd).start()

        def l_wait(j, bank):
            ix, dv, si, sd = bank
            b = ebase + j * SC_CHUNK
            pltpu.make_async_copy(dst_hbm.at[pl.ds(b, SC_CHUNK)], ix,
                                  si).wait()
            pltpu.make_async_copy(
                p_hbm.at[pl.ds(b, SC_CHUNK), pl.ds(f0, FB)], dv, sd).wait()

        pltpu.sync_copy(z_hbm.at[pl.ds(rbase, RPT)], q_sh.at[pl.ds(rbase, RPT)])
        plsc.subcore_barrier()

        for b in range(4):
            l_issue(b, banks[b])

        def group(g, carry):
            for b in range(4):
                j = 4 * g + b
                l_wait(j, banks[b])
                pltpu.async_copy(banks[b][1], q_sh.at[banks[b][0]], sa,
                                 add=True)
            for b in range(4):
                pltpu.make_async_copy(banks[b][1], q_sh.at[banks[b][0]],
                                      sa).wait()

            @pl.when(g < ngrp - 1)
            def _():
                for b in range(4):
                    l_issue(4 * (g + 1) + b, banks[b])

            return carry

        lax.fori_loop(0, ngrp, group, 0)
        l_issue(nch - 1, banks[0])
        l_wait(nch - 1, banks[0])
        pltpu.sync_copy(banks[0][1], q_sh.at[banks[0][0]], add=True)
        plsc.subcore_barrier()
        pltpu.sync_copy(q_sh.at[pl.ds(rbase, RPT)],
                        q_hbm.at[pl.ds(rbase, RPT), pl.ds(f0, FB)])
        plsc.subcore_barrier()


def _sc_scatter(p, dst, zeros):
    return pl.kernel(
        _scatter_body,
        out_type=jax.ShapeDtypeStruct((NP, HD), jnp.float32),
        mesh=_mesh,
        scratch_types=[
            pltpu.VMEM_SHARED((NP, FB), jnp.float32),
            pltpu.VMEM((SC_CHUNK,), jnp.int32),
            pltpu.VMEM((SC_CHUNK, FB), jnp.float32),
            pltpu.VMEM((SC_CHUNK,), jnp.int32),
            pltpu.VMEM((SC_CHUNK, FB), jnp.float32),
            pltpu.VMEM((SC_CHUNK,), jnp.int32),
            pltpu.VMEM((SC_CHUNK, FB), jnp.float32),
            pltpu.VMEM((SC_CHUNK,), jnp.int32),
            pltpu.VMEM((SC_CHUNK, FB), jnp.float32),
            pltpu.SemaphoreType.DMA,
            pltpu.SemaphoreType.DMA,
            pltpu.SemaphoreType.DMA,
            pltpu.SemaphoreType.DMA,
            pltpu.SemaphoreType.DMA,
            pltpu.SemaphoreType.DMA,
            pltpu.SemaphoreType.DMA,
            pltpu.SemaphoreType.DMA,
            pltpu.SemaphoreType.DMA,
        ],
    )(p, dst, zeros)


def _combine_body(h0b_hbm, q_hbm, p_hbm, src_hbm, rev_hbm, out_hbm,
                  isrc, irev, ha0, qa0, pa0, ha1, qa1, pa1,
                  sh0, sq0, sp0, sh1, sq1, sp1):
    base = _wid() * EPW
    pltpu.sync_copy(src_hbm.at[pl.ds(base, EPW)], isrc)
    pltpu.sync_copy(rev_hbm.at[pl.ds(base, EPW)], irev)
    nch = EPW // CC

    def issue(j, hv, qv, pv, sh, sq, sp, n=CC):
        o = j * CC
        pltpu.make_async_copy(h0b_hbm.at[pl.ds(base + o, n)],
                              hv.at[pl.ds(0, n)], sh).start()
        pltpu.make_async_copy(q_hbm.at[isrc.at[pl.ds(o, n)]],
                              qv.at[pl.ds(0, n)], sq).start()
        pltpu.make_async_copy(p_hbm.at[irev.at[pl.ds(o, n)]],
                              pv.at[pl.ds(0, n)], sp).start()

    def wait(j, hv, qv, pv, sh, sq, sp, n=CC):
        o = j * CC
        pltpu.make_async_copy(h0b_hbm.at[pl.ds(base + o, n)],
                              hv.at[pl.ds(0, n)], sh).wait()
        pltpu.make_async_copy(q_hbm.at[isrc.at[pl.ds(o, n)]],
                              qv.at[pl.ds(0, n)], sq).wait()
        pltpu.make_async_copy(p_hbm.at[irev.at[pl.ds(o, n)]],
                              pv.at[pl.ds(0, n)], sp).wait()

    def compute_store(j, hv, qv, pv, n=CC):
        def row(r, c2):
            for g in range(HD // 16):
                sl = (r, pl.ds(g * 16, 16))
                hv[sl] = jnp.maximum(hv[sl] + qv[sl] - pv[sl], 0.0)
            return c2

        lax.fori_loop(0, n, row, 0)
        pltpu.sync_copy(hv.at[pl.ds(0, n)],
                        out_hbm.at[pl.ds(base + j * CC, n)])

    issue(0, ha0, qa0, pa0, sh0, sq0, sp0)
    issue(1, ha1, qa1, pa1, sh1, sq1, sp1)

    def pair(k, carry):
        j0 = 2 * k
        wait(j0, ha0, qa0, pa0, sh0, sq0, sp0)
        compute_store(j0, ha0, qa0, pa0)

        @pl.when(k < nch // 2 - 1)
        def _():
            issue(j0 + 2, ha0, qa0, pa0, sh0, sq0, sp0)

        wait(j0 + 1, ha1, qa1, pa1, sh1, sq1, sp1)
        compute_store(j0 + 1, ha1, qa1, pa1)

        @pl.when(k < nch // 2 - 1)
        def _():
            issue(j0 + 3, ha1, qa1, pa1, sh1, sq1, sp1)

        return carry

    lax.fori_loop(0, nch // 2, pair, 0)
    issue(nch, ha0, qa0, pa0, sh0, sq0, sp0, n=GT)
    wait(nch, ha0, qa0, pa0, sh0, sq0, sp0, n=GT)
    compute_store(nch, ha0, qa0, pa0, n=GT)


def _sc_combine(h0b, q, p, src, rev):
    return pl.kernel(
        _combine_body,
        out_type=jax.ShapeDtypeStruct((E, HD), jnp.float32),
        mesh=_mesh,
        scratch_types=[
            pltpu.VMEM((EPW,), jnp.int32),
            pltpu.VMEM((EPW,), jnp.int32),
            pltpu.VMEM((CC, HD), jnp.float32),
            pltpu.VMEM((CC, HD), jnp.float32),
            pltpu.VMEM((CC, HD), jnp.float32),
            pltpu.VMEM((CC, HD), jnp.float32),
            pltpu.VMEM((CC, HD), jnp.float32),
            pltpu.VMEM((CC, HD), jnp.float32),
            pltpu.SemaphoreType.DMA,
            pltpu.SemaphoreType.DMA,
            pltpu.SemaphoreType.DMA,
            pltpu.SemaphoreType.DMA,
            pltpu.SemaphoreType.DMA,
            pltpu.SemaphoreType.DMA,
        ],
    )(h0b, q, p, src, rev)


def _tx_body(x_ref, w_ref, b_ref, o_ref):
    o_ref[...] = jnp.dot(x_ref[...], w_ref[...],
                         preferred_element_type=jnp.float32) + b_ref[...]


def _tc_tx(x, wixT, bi):
    bn = 1000
    return pl.pallas_call(
        _tx_body,
        grid=(N // bn,),
        in_specs=[
            pl.BlockSpec((bn, D), lambda i: (i, 0)),
            pl.BlockSpec((D, HD), lambda i: (0, 0)),
            pl.BlockSpec((1, HD), lambda i: (0, 0)),
        ],
        out_specs=pl.BlockSpec((bn, HD), lambda i: (i, 0)),
        out_shape=jax.ShapeDtypeStruct((N, HD), jnp.float32),
    )(x, wixT, bi)


def _kb_body(txs_ref, ea_ref, wie_ref, wh_ref, bh_ref, h0b_ref, p0_ref):
    h0 = txs_ref[...] + jnp.dot(
        ea_ref[...].astype(jnp.bfloat16), wie_ref[...],
        preferred_element_type=jnp.float32)
    p0_ref[...] = jnp.dot(jnp.maximum(h0, 0.0).astype(jnp.bfloat16),
                          wh_ref[...], preferred_element_type=jnp.float32)
    h0b_ref[...] = h0 + bh_ref[...]


def _tc_kb(txsrc, ea, wieT, whT, bh):
    be = 640
    return pl.pallas_call(
        _kb_body,
        grid=(E // be,),
        in_specs=[
            pl.BlockSpec((be, HD), lambda i: (i, 0)),
            pl.BlockSpec((be, EB), lambda i: (i, 0)),
            pl.BlockSpec((EB, HD), lambda i: (0, 0)),
            pl.BlockSpec((HD, HD), lambda i: (0, 0)),
            pl.BlockSpec((1, HD), lambda i: (0, 0)),
        ],
        out_specs=[
            pl.BlockSpec((be, HD), lambda i: (i, 0)),
            pl.BlockSpec((be, HD), lambda i: (i, 0)),
        ],
        out_shape=[
            jax.ShapeDtypeStruct((E, HD), jnp.float32),
            jax.ShapeDtypeStruct((E, HD), jnp.float32),
        ],
    )(txsrc, ea, wieT, whT, bh)


def _mm_body(h_ref, w_ref, o_ref):
    o_ref[...] = jnp.dot(h_ref[...].astype(jnp.bfloat16), w_ref[...],
                         preferred_element_type=jnp.float32)


def _tc_mm(hh, whT):
    be = 640
    return pl.pallas_call(
        _mm_body,
        grid=(E // be,),
        in_specs=[
            pl.BlockSpec((be, HD), lambda i: (i, 0)),
            pl.BlockSpec((HD, HD), lambda i: (0, 0)),
        ],
        out_specs=pl.BlockSpec((be, HD), lambda i: (i, 0)),
        out_shape=jax.ShapeDtypeStruct((E, HD), jnp.float32),
    )(hh, whT)


def _k4_body(mn_ref, x_ref, wt_ref, wox_ref, wom_ref, bt_ref, bo_ref, o_ref):
    mn = mn_ref[...]
    xb = x_ref[...].astype(jnp.bfloat16)
    tx = jnp.dot(xb, wt_ref[...], preferred_element_type=jnp.float32) + bt_ref[...]
    rowsum = jnp.sum(mn, axis=1, keepdims=True)
    mnp = jnp.where(rowsum == 0.0, tx, mn)
    acc = jnp.dot(xb, wox_ref[...], preferred_element_type=jnp.float32)
    acc = acc + jnp.dot(mnp.astype(jnp.bfloat16), wom_ref[...],
                        preferred_element_type=jnp.float32)
    o_ref[...] = jnp.maximum(acc + bo_ref[...], 0.0)


def _tc_k4(mn, x, wtT, woxT, womT, bt, bo):
    bn = 1000
    return pl.pallas_call(
        _k4_body,
        grid=(N // bn,),
        in_specs=[
            pl.BlockSpec((bn, HD), lambda i: (i, 0)),
            pl.BlockSpec((bn, D), lambda i: (i, 0)),
            pl.BlockSpec((D, HD), lambda i: (0, 0)),
            pl.BlockSpec((D, HD), lambda i: (0, 0)),
            pl.BlockSpec((HD, HD), lambda i: (0, 0)),
            pl.BlockSpec((1, HD), lambda i: (0, 0)),
            pl.BlockSpec((1, HD), lambda i: (0, 0)),
        ],
        out_specs=pl.BlockSpec((bn, HD), lambda i: (i, 0)),
        out_shape=jax.ShapeDtypeStruct((N, HD), jnp.float32),
    )(mn, x, wtT, woxT, womT, bt, bo)


def kernel(x, edge_index, edge_attr, rev_edge_index,
           Wi0, bi0, Wh0, bh0, Wo0, bo0, Wt0, bt0,
           Wi1, bi1, Wh1, bh1, Wo1, bo1, Wt1, bt1):
    src = edge_index[0]
    dst = edge_index[1]
    rev = rev_edge_index

    wixT = Wi1[:, :D].T
    wieT = Wi1[:, D:].T.astype(jnp.bfloat16)
    whT = Wh1.T.astype(jnp.bfloat16)
    wtT = Wt1.T.astype(jnp.bfloat16)
    woxT = Wo1[:, :D].T.astype(jnp.bfloat16)
    womT = Wo1[:, D:].T.astype(jnp.bfloat16)
    bi = bi1.reshape(1, HD)
    bh = bh1.reshape(1, HD)
    bt = bt1.reshape(1, HD)
    bo = bo1.reshape(1, HD)
    zeros = jnp.zeros((NP, FB), jnp.float32)

    tx = _tc_tx(x, wixT, bi)
    txsrc = _sc_gather(tx, src)
    h0b, p = _tc_kb(txsrc, edge_attr, wieT, whT, bh)

    for it in range(2):
        q = _sc_scatter(p, dst, zeros)
        hh = _sc_combine(h0b, q, p, src, rev)
        if it == 0:
            p = _tc_mm(hh, whT)

    mn = _sc_scatter(hh, dst, zeros)
    return _tc_k4(mn[:N], x, wtT, woxT, womT, bt, bo)


# speedup vs baseline: 1.0358x; 1.0358x over previous
import functools

import jax
import jax.numpy as jnp
from jax import lax
from jax.experimental import pallas as pl
from jax.experimental.pallas import tpu as pltpu
from jax.experimental.pallas import tpu_sc as plsc

N = 10000
E = 160000
D = 256
EB = 16
HD = 512

NP = 10240
NC = 2
NS = 16
NW = NC * NS
EPW = E // NW
EPT = E // NS

GC = 32
GT = 8
CC = 32
SC_CHUNK = 80
FB = 128
RPT = NP // NS

_mesh = plsc.VectorSubcoreMesh(core_axis_name="c", subcore_axis_name="s")


def _wid():
    return lax.axis_index("s") * NC + lax.axis_index("c")


def _gather_body(table_hbm, idx_hbm, out_hbm, idx_all, rv0, rv1, sg0, sg1,
                 so0, so1):
    base = _wid() * EPW
    pltpu.sync_copy(idx_hbm.at[pl.ds(base, EPW)], idx_all)
    nch = EPW // GC

    def g_issue(j, rv, sg, n=GC):
        pltpu.make_async_copy(
            table_hbm.at[idx_all.at[pl.ds(j * GC, n)]], rv.at[pl.ds(0, n)],
            sg).start()

    def g_wait(j, rv, sg, n=GC):
        pltpu.make_async_copy(
            table_hbm.at[idx_all.at[pl.ds(j * GC, n)]], rv.at[pl.ds(0, n)],
            sg).wait()

    def s_issue(j, rv, so, n=GC):
        pltpu.make_async_copy(
            rv.at[pl.ds(0, n)], out_hbm.at[pl.ds(base + j * GC, n)],
            so).start()

    def s_wait(j, rv, so, n=GC):
        pltpu.make_async_copy(
            rv.at[pl.ds(0, n)], out_hbm.at[pl.ds(base + j * GC, n)],
            so).wait()

    g_issue(0, rv0, sg0)
    g_issue(1, rv1, sg1)

    def pair(k, carry):
        j0 = 2 * k
        g_wait(j0, rv0, sg0)
        s_issue(j0, rv0, so0)
        g_wait(j0 + 1, rv1, sg1)
        s_issue(j0 + 1, rv1, so1)
        s_wait(j0, rv0, so0)

        @pl.when(k < nch // 2 - 1)
        def _():
            g_issue(j0 + 2, rv0, sg0)

        s_wait(j0 + 1, rv1, so1)

        @pl.when(k < nch // 2 - 1)
        def _():
            g_issue(j0 + 3, rv1, sg1)

        return carry

    lax.fori_loop(0, nch // 2, pair, 0)
    g_issue(nch, rv0, sg0, n=GT)
    g_wait(nch, rv0, sg0, n=GT)
    s_issue(nch, rv0, so0, n=GT)
    s_wait(nch, rv0, so0, n=GT)


def _sc_gather(table, idx):
    return pl.kernel(
        _gather_body,
        out_type=jax.ShapeDtypeStruct((E, HD), jnp.float32),
        mesh=_mesh,
        scratch_types=[
            pltpu.VMEM((EPW,), jnp.int32),
            pltpu.VMEM((GC, HD), jnp.float32),
            pltpu.VMEM((GC, HD), jnp.float32),
            pltpu.SemaphoreType.DMA,
            pltpu.SemaphoreType.DMA,
            pltpu.SemaphoreType.DMA,
            pltpu.SemaphoreType.DMA,
        ],
    )(table, idx)


def _scatter_body(p_hbm, dst_hbm, z_hbm, q_hbm, q_sh, ix0, dv0, ix1, dv1,
                  si0, sd0, si1, sd1):
    cid = lax.axis_index("c")
    sid = lax.axis_index("s")
    ebase = sid * EPT
    rbase = sid * RPT
    nch = EPT // SC_CHUNK

    for p in range(2):
        f0 = (cid * 2 + p) * FB

        def l_issue(j, ix, dv, si, sd):
            b = ebase + j * SC_CHUNK
            pltpu.make_async_copy(dst_hbm.at[pl.ds(b, SC_CHUNK)], ix,
                                  si).start()
            pltpu.make_async_copy(
                p_hbm.at[pl.ds(b, SC_CHUNK), pl.ds(f0, FB)], dv, sd).start()

        def l_wait(j, ix, dv, si, sd):
            b = ebase + j * SC_CHUNK
            pltpu.make_async_copy(dst_hbm.at[pl.ds(b, SC_CHUNK)], ix,
                                  si).wait()
            pltpu.make_async_copy(
                p_hbm.at[pl.ds(b, SC_CHUNK), pl.ds(f0, FB)], dv, sd).wait()

        pltpu.sync_copy(z_hbm.at[pl.ds(rbase, RPT)], q_sh.at[pl.ds(rbase, RPT)])
        plsc.subcore_barrier()

        l_issue(0, ix0, dv0, si0, sd0)
        l_wait(0, ix0, dv0, si0, sd0)
        pltpu.sync_copy(dv0, q_sh.at[ix0], add=True)
        l_issue(1, ix0, dv0, si0, sd0)
        l_issue(2, ix1, dv1, si1, sd1)

        def pair(k, carry):
            j0 = 2 * k + 1
            l_wait(j0, ix0, dv0, si0, sd0)
            pltpu.sync_copy(dv0, q_sh.at[ix0], add=True)

            @pl.when(k < (nch - 1) // 2 - 1)
            def _():
                l_issue(j0 + 2, ix0, dv0, si0, sd0)

            l_wait(j0 + 1, ix1, dv1, si1, sd1)
            pltpu.sync_copy(dv1, q_sh.at[ix1], add=True)

            @pl.when(k < (nch - 1) // 2 - 1)
            def _():
                l_issue(j0 + 3, ix1, dv1, si1, sd1)

            return carry

        lax.fori_loop(0, (nch - 1) // 2, pair, 0)
        plsc.subcore_barrier()
        pltpu.sync_copy(q_sh.at[pl.ds(rbase, RPT)],
                        q_hbm.at[pl.ds(rbase, RPT), pl.ds(f0, FB)])
        plsc.subcore_barrier()


def _sc_scatter(p, dst, zeros):
    return pl.kernel(
        _scatter_body,
        out_type=jax.ShapeDtypeStruct((NP, HD), jnp.float32),
        mesh=_mesh,
        scratch_types=[
            pltpu.VMEM_SHARED((NP, FB), jnp.float32),
            pltpu.VMEM((SC_CHUNK,), jnp.int32),
            pltpu.VMEM((SC_CHUNK, FB), jnp.float32),
            pltpu.VMEM((SC_CHUNK,), jnp.int32),
            pltpu.VMEM((SC_CHUNK, FB), jnp.float32),
            pltpu.SemaphoreType.DMA,
            pltpu.SemaphoreType.DMA,
            pltpu.SemaphoreType.DMA,
            pltpu.SemaphoreType.DMA,
        ],
    )(p, dst, zeros)


def _combine_body(h0b_hbm, q_hbm, p_hbm, src_hbm, rev_hbm, out_hbm,
                  isrc, irev, ha0, qa0, pa0, ha1, qa1, pa1,
                  sh0, sq0, sp0, sh1, sq1, sp1):
    base = _wid() * EPW
    pltpu.sync_copy(src_hbm.at[pl.ds(base, EPW)], isrc)
    pltpu.sync_copy(rev_hbm.at[pl.ds(base, EPW)], irev)
    nch = EPW // CC

    def issue(j, hv, qv, pv, sh, sq, sp, n=CC):
        o = j * CC
        pltpu.make_async_copy(h0b_hbm.at[pl.ds(base + o, n)],
                              hv.at[pl.ds(0, n)], sh).start()
        pltpu.make_async_copy(q_hbm.at[isrc.at[pl.ds(o, n)]],
                              qv.at[pl.ds(0, n)], sq).start()
        pltpu.make_async_copy(p_hbm.at[irev.at[pl.ds(o, n)]],
                              pv.at[pl.ds(0, n)], sp).start()

    def wait(j, hv, qv, pv, sh, sq, sp, n=CC):
        o = j * CC
        pltpu.make_async_copy(h0b_hbm.at[pl.ds(base + o, n)],
                              hv.at[pl.ds(0, n)], sh).wait()
        pltpu.make_async_copy(q_hbm.at[isrc.at[pl.ds(o, n)]],
                              qv.at[pl.ds(0, n)], sq).wait()
        pltpu.make_async_copy(p_hbm.at[irev.at[pl.ds(o, n)]],
                              pv.at[pl.ds(0, n)], sp).wait()

    def compute_store(j, hv, qv, pv, n=CC):
        def row(r, c2):
            for g in range(HD // 16):
                sl = (r, pl.ds(g * 16, 16))
                hv[sl] = jnp.maximum(hv[sl] + qv[sl] - pv[sl], 0.0)
            return c2

        lax.fori_loop(0, n, row, 0)
        pltpu.sync_copy(hv.at[pl.ds(0, n)],
                        out_hbm.at[pl.ds(base + j * CC, n)])

    issue(0, ha0, qa0, pa0, sh0, sq0, sp0)
    issue(1, ha1, qa1, pa1, sh1, sq1, sp1)

    def pair(k, carry):
        j0 = 2 * k
        wait(j0, ha0, qa0, pa0, sh0, sq0, sp0)
        compute_store(j0, ha0, qa0, pa0)

        @pl.when(k < nch // 2 - 1)
        def _():
            issue(j0 + 2, ha0, qa0, pa0, sh0, sq0, sp0)

        wait(j0 + 1, ha1, qa1, pa1, sh1, sq1, sp1)
        compute_store(j0 + 1, ha1, qa1, pa1)

        @pl.when(k < nch // 2 - 1)
        def _():
            issue(j0 + 3, ha1, qa1, pa1, sh1, sq1, sp1)

        return carry

    lax.fori_loop(0, nch // 2, pair, 0)
    issue(nch, ha0, qa0, pa0, sh0, sq0, sp0, n=GT)
    wait(nch, ha0, qa0, pa0, sh0, sq0, sp0, n=GT)
    compute_store(nch, ha0, qa0, pa0, n=GT)


def _sc_combine(h0b, q, p, src, rev):
    return pl.kernel(
        _combine_body,
        out_type=jax.ShapeDtypeStruct((E, HD), jnp.float32),
        mesh=_mesh,
        scratch_types=[
            pltpu.VMEM((EPW,), jnp.int32),
            pltpu.VMEM((EPW,), jnp.int32),
            pltpu.VMEM((CC, HD), jnp.float32),
            pltpu.VMEM((CC, HD), jnp.float32),
            pltpu.VMEM((CC, HD), jnp.float32),
            pltpu.VMEM((CC, HD), jnp.float32),
            pltpu.VMEM((CC, HD), jnp.float32),
            pltpu.VMEM((CC, HD), jnp.float32),
            pltpu.SemaphoreType.DMA,
            pltpu.SemaphoreType.DMA,
            pltpu.SemaphoreType.DMA,
            pltpu.SemaphoreType.DMA,
            pltpu.SemaphoreType.DMA,
            pltpu.SemaphoreType.DMA,
        ],
    )(h0b, q, p, src, rev)


def _tx_body(x_ref, w_ref, b_ref, o_ref):
    o_ref[...] = jnp.dot(x_ref[...], w_ref[...],
                         preferred_element_type=jnp.float32) + b_ref[...]


def _tc_tx(x, wixT, bi):
    bn = 1000
    return pl.pallas_call(
        _tx_body,
        grid=(N // bn,),
        in_specs=[
            pl.BlockSpec((bn, D), lambda i: (i, 0)),
            pl.BlockSpec((D, HD), lambda i: (0, 0)),
            pl.BlockSpec((1, HD), lambda i: (0, 0)),
        ],
        out_specs=pl.BlockSpec((bn, HD), lambda i: (i, 0)),
        out_shape=jax.ShapeDtypeStruct((N, HD), jnp.float32),
    )(x, wixT, bi)


def _kb_body(txs_ref, ea_ref, wie_ref, wh_ref, bh_ref, h0b_ref, p0_ref):
    h0 = txs_ref[...] + jnp.dot(
        ea_ref[...].astype(jnp.bfloat16), wie_ref[...],
        preferred_element_type=jnp.float32)
    p0_ref[...] = jnp.dot(jnp.maximum(h0, 0.0).astype(jnp.bfloat16),
                          wh_ref[...], preferred_element_type=jnp.float32)
    h0b_ref[...] = h0 + bh_ref[...]


def _tc_kb(txsrc, ea, wieT, whT, bh):
    be = 640
    return pl.pallas_call(
        _kb_body,
        grid=(E // be,),
        in_specs=[
            pl.BlockSpec((be, HD), lambda i: (i, 0)),
            pl.BlockSpec((be, EB), lambda i: (i, 0)),
            pl.BlockSpec((EB, HD), lambda i: (0, 0)),
            pl.BlockSpec((HD, HD), lambda i: (0, 0)),
            pl.BlockSpec((1, HD), lambda i: (0, 0)),
        ],
        out_specs=[
            pl.BlockSpec((be, HD), lambda i: (i, 0)),
            pl.BlockSpec((be, HD), lambda i: (i, 0)),
        ],
        out_shape=[
            jax.ShapeDtypeStruct((E, HD), jnp.float32),
            jax.ShapeDtypeStruct((E, HD), jnp.float32),
        ],
    )(txsrc, ea, wieT, whT, bh)


def _mm_body(h_ref, w_ref, o_ref):
    o_ref[...] = jnp.dot(h_ref[...].astype(jnp.bfloat16), w_ref[...],
                         preferred_element_type=jnp.float32)


def _tc_mm(hh, whT):
    be = 640
    return pl.pallas_call(
        _mm_body,
        grid=(E // be,),
        in_specs=[
            pl.BlockSpec((be, HD), lambda i: (i, 0)),
            pl.BlockSpec((HD, HD), lambda i: (0, 0)),
        ],
        out_specs=pl.BlockSpec((be, HD), lambda i: (i, 0)),
        out_shape=jax.ShapeDtypeStruct((E, HD), jnp.float32),
    )(hh, whT)


def _k4_body(mn_ref, x_ref, wt_ref, wox_ref, wom_ref, bt_ref, bo_ref, o_ref):
    mn = mn_ref[...]
    xb = x_ref[...].astype(jnp.bfloat16)
    tx = jnp.dot(xb, wt_ref[...], preferred_element_type=jnp.float32) + bt_ref[...]
    rowsum = jnp.sum(mn, axis=1, keepdims=True)
    mnp = jnp.where(rowsum == 0.0, tx, mn)
    acc = jnp.dot(xb, wox_ref[...], preferred_element_type=jnp.float32)
    acc = acc + jnp.dot(mnp.astype(jnp.bfloat16), wom_ref[...],
                        preferred_element_type=jnp.float32)
    o_ref[...] = jnp.maximum(acc + bo_ref[...], 0.0)


def _tc_k4(mn, x, wtT, woxT, womT, bt, bo):
    bn = 1000
    return pl.pallas_call(
        _k4_body,
        grid=(N // bn,),
        in_specs=[
            pl.BlockSpec((bn, HD), lambda i: (i, 0)),
            pl.BlockSpec((bn, D), lambda i: (i, 0)),
            pl.BlockSpec((D, HD), lambda i: (0, 0)),
            pl.BlockSpec((D, HD), lambda i: (0, 0)),
            pl.BlockSpec((HD, HD), lambda i: (0, 0)),
            pl.BlockSpec((1, HD), lambda i: (0, 0)),
            pl.BlockSpec((1, HD), lambda i: (0, 0)),
        ],
        out_specs=pl.BlockSpec((bn, HD), lambda i: (i, 0)),
        out_shape=jax.ShapeDtypeStruct((N, HD), jnp.float32),
    )(mn, x, wtT, woxT, womT, bt, bo)


def kernel(x, edge_index, edge_attr, rev_edge_index,
           Wi0, bi0, Wh0, bh0, Wo0, bo0, Wt0, bt0,
           Wi1, bi1, Wh1, bh1, Wo1, bo1, Wt1, bt1):
    src = edge_index[0]
    dst = edge_index[1]
    rev = rev_edge_index

    wixT = Wi1[:, :D].T
    wieT = Wi1[:, D:].T.astype(jnp.bfloat16)
    whT = Wh1.T.astype(jnp.bfloat16)
    wtT = Wt1.T.astype(jnp.bfloat16)
    woxT = Wo1[:, :D].T.astype(jnp.bfloat16)
    womT = Wo1[:, D:].T.astype(jnp.bfloat16)
    bi = bi1.reshape(1, HD)
    bh = bh1.reshape(1, HD)
    bt = bt1.reshape(1, HD)
    bo = bo1.reshape(1, HD)
    zeros = jnp.zeros((NP, FB), jnp.float32)

    tx = _tc_tx(x, wixT, bi)
    txsrc = _sc_gather(tx, src)
    h0b, p = _tc_kb(txsrc, edge_attr, wieT, whT, bh)

    for it in range(2):
        q = _sc_scatter(p, dst, zeros)
        hh = _sc_combine(h0b, q, p, src, rev)
        if it == 0:
            p = _tc_mm(hh, whT)

    mn = _sc_scatter(hh, dst, zeros)
    return _tc_k4(mn[:N], x, wtT, woxT, womT, bt, bo)
